# baseline (speedup 1.0000x reference)
import functools

import jax
import jax.numpy as jnp
from jax import lax
from jax.experimental import pallas as pl
from jax.experimental.pallas import tpu as pltpu
from jax.experimental.pallas import tpu_sc as plsc

B, T, D, DFF = 4, 4096, 2048, 8192
K = 512
CHUNK = 512
NCHUNK = (B * T) // CHUNK
RC = T // CHUNK
NTOK = B * K
AUX_COEF = 0.01
CAPACITY = 0.125

SC_CORES = 2
SC_SUBCORES = 16
NW = SC_CORES * SC_SUBCORES
PER_W = NTOK // NW
ROWCH = 32

_HI = 0x7FFFFFFF
_SIGN = -2**31


def _flat_cumsum(m2, u_tri, s_blk):
    incl = lax.dot_general(m2, u_tri, (((1,), (0,)), ((), ())),
                           precision=lax.Precision.HIGHEST,
                           preferred_element_type=jnp.float32)
    rowtot = jnp.sum(m2, axis=1, keepdims=True)
    off = lax.dot_general(s_blk, rowtot, (((1,), (0,)), ((), ())),
                          precision=lax.Precision.HIGHEST,
                          preferred_element_type=jnp.float32)
    return incl + off


def _router_body(x_ref, wr_ref, idx_ref, rw_ref, aux_ref, scores_ref):
    i = pl.program_id(0)
    s = lax.dot_general(wr_ref[...].astype(jnp.bfloat16),
                        x_ref[...].astype(jnp.bfloat16),
                        (((1,), (1,)), ((), ())),
                        preferred_element_type=jnp.float32)
    scores_ref[pl.ds(i, 1), :] = s

    @pl.when(i == NCHUNK - 1)
    def _finish():
        s2 = scores_ref[...]
        s3 = s2.reshape(B, RC, CHUNK)
        bits = lax.bitcast_convert_type(s3, jnp.int32)
        key = jnp.where(bits < 0, bits ^ jnp.int32(_HI), bits)

        def _count_ge(thr):
            m = (key >= thr).astype(jnp.int32)
            return jnp.sum(jnp.sum(m, axis=2, keepdims=True), axis=1,
                           keepdims=True)

        def _bit_step(it, prefix_bits):
            cand = prefix_bits | lax.shift_left(jnp.int32(1), 31 - it)
            cnt = _count_ge(cand ^ jnp.int32(_SIGN))
            return jnp.where(cnt >= K, cand, prefix_bits)

        prefix = lax.fori_loop(0, 32, _bit_step,
                               jnp.zeros((B, 1, 1), jnp.int32))
        thr = prefix ^ jnp.int32(_SIGN)
        gt = key > thr
        eq = key == thr
        cnt_gt = jnp.sum(jnp.sum(gt.astype(jnp.int32), axis=2, keepdims=True),
                         axis=1, keepdims=True)
        needed = (K - cnt_gt).astype(jnp.float32)

        lane_i = lax.broadcasted_iota(jnp.int32, (CHUNK, CHUNK), 0)
        lane_ti = lax.broadcasted_iota(jnp.int32, (CHUNK, CHUNK), 1)
        u_tri = (lane_i <= lane_ti).astype(jnp.float32)
        ra = lax.broadcasted_iota(jnp.int32, (NCHUNK, NCHUNK), 0)
        rb = lax.broadcasted_iota(jnp.int32, (NCHUNK, NCHUNK), 1)
        s_blk = ((rb < ra) & ((rb >> 3) == (ra >> 3))).astype(jnp.float32)

        eq2 = eq.reshape(NCHUNK, CHUNK).astype(jnp.float32)
        eq_rank = _flat_cumsum(eq2, u_tri, s_blk).reshape(B, RC, CHUNK)
        selected = gt | (eq & (eq_rank <= needed))
        sel2 = selected.reshape(NCHUNK, CHUNK).astype(jnp.float32)
        sel_cum = _flat_cumsum(sel2, u_tri, s_blk)
        sel_rank = jnp.where(sel2 > 0.0, sel_cum, jnp.float32(0.0))

        jj = (lane_i + 1).astype(jnp.float32)
        lane_row = lax.broadcasted_iota(jnp.int32,
                                        (1, CHUNK), 1).astype(jnp.float32)
        for b in range(B):
            acc = jnp.zeros((2, CHUNK), jnp.float32)
            for r in range(RC):
                rr = b * RC + r
                m_jc = (jj == sel_rank[rr:rr + 1, :]).astype(jnp.float32)
                tid = lane_row + jnp.float32(r * CHUNK + b * T)
                v = jnp.concatenate([tid, s2[rr:rr + 1, :]], axis=0)
                acc = acc + lax.dot_general(
                    v, m_jc, (((1,), (1,)), ((), ())),
                    precision=lax.Precision.HIGHEST,
                    preferred_element_type=jnp.float32)
            idx_ref[b:b + 1, :] = acc[0:1, :].astype(jnp.int32)
            sr = acc[1:2, :]
            e = jnp.exp(sr - jnp.max(sr, axis=1, keepdims=True))
            rw_ref[b:b + 1, :] = e / jnp.sum(e, axis=1, keepdims=True)

        probs = 1.0 / (1.0 + jnp.exp(-s3))
        frac = jnp.sum(jnp.sum(probs, axis=2, keepdims=True), axis=1,
                       keepdims=True) * jnp.float32(1.0 / T)
        dev = (frac - jnp.float32(CAPACITY)) ** 2
        aux_ref[...] = jnp.sum(dev.reshape(B, 1), axis=0,
                               keepdims=True) * jnp.float32(AUX_COEF / B)


def _router(x_flat, wr_row):
    return pl.pallas_call(
        _router_body,
        grid=(NCHUNK,),
        in_specs=[
            pl.BlockSpec((CHUNK, D), lambda i: (i, 0)),
            pl.BlockSpec((1, D), lambda i: (0, 0)),
        ],
        out_specs=[
            pl.BlockSpec((B, K), lambda i: (0, 0)),
            pl.BlockSpec((B, K), lambda i: (0, 0)),
            pl.BlockSpec((1, 1), lambda i: (0, 0)),
        ],
        out_shape=[
            jax.ShapeDtypeStruct((B, K), jnp.int32),
            jax.ShapeDtypeStruct((B, K), jnp.float32),
            jax.ShapeDtypeStruct((1, 1), jnp.float32),
        ],
        scratch_shapes=[pltpu.VMEM((NCHUNK, CHUNK), jnp.float32)],
        compiler_params=pltpu.CompilerParams(
            dimension_semantics=("arbitrary",)),
    )(x_flat, wr_row)


def _sc_worker_id():
    return lax.axis_index("s") * SC_CORES + lax.axis_index("c")


def _gather_body(x_hbm, idx_hbm, out_hbm, idxv, rows, sem):
    base = _sc_worker_id() * PER_W
    pltpu.sync_copy(idx_hbm.at[pl.ds(base, PER_W)], idxv)
    for c in range(PER_W // ROWCH):
        pltpu.async_copy(x_hbm.at[idxv.at[pl.ds(c * ROWCH, ROWCH)]], rows,
                         sem).wait()
        pltpu.sync_copy(rows, out_hbm.at[pl.ds(base + c * ROWCH, ROWCH)])


def _gather(x_flat, idx_flat):
    mesh = plsc.VectorSubcoreMesh(core_axis_name="c", subcore_axis_name="s",
                                  num_cores=SC_CORES, num_subcores=SC_SUBCORES)
    return pl.kernel(
        _gather_body,
        out_type=jax.ShapeDtypeStruct((NTOK, D), jnp.float32),
        mesh=mesh,
        scratch_types=[
            pltpu.VMEM((PER_W,), jnp.int32),
            pltpu.VMEM((ROWCH, D), jnp.float32),
            pltpu.SemaphoreType.DMA,
        ],
    )(x_flat, idx_flat)


CPW = (B * T) // NW
CCH = 16


def _copy_body(x_hbm, dep_hbm, out_hbm, sem):
    base = _sc_worker_id() * CPW
    pltpu.async_copy(x_hbm.at[pl.ds(base, CPW)],
                     out_hbm.at[pl.ds(base, CPW)], sem).wait()


def _sc_copy(x_flat, dep):
    mesh = plsc.VectorSubcoreMesh(core_axis_name="c", subcore_axis_name="s",
                                  num_cores=SC_CORES, num_subcores=SC_SUBCORES)
    return pl.kernel(
        _copy_body,
        out_type=jax.ShapeDtypeStruct((B * T, D), jnp.float32),
        mesh=mesh,
        scratch_types=[pltpu.SemaphoreType.DMA],
    )(x_flat, dep)


def _scatter_body(w_hbm, idx_hbm, out_hbm, idxv2, rows, sem):
    base = _sc_worker_id() * PER_W
    for c in range(PER_W // ROWCH):
        pltpu.sync_copy(idx_hbm.at[pl.ds(base + c * ROWCH, ROWCH)],
                        idxv2.at[c])
        pltpu.sync_copy(w_hbm.at[pl.ds(base + c * ROWCH, ROWCH)], rows)
        pltpu.async_copy(rows, out_hbm.at[idxv2.at[c]], sem).wait()


def _scatter(weighted, idx_flat, out_ref):
    mesh = plsc.VectorSubcoreMesh(core_axis_name="c", subcore_axis_name="s",
                                  num_cores=SC_CORES, num_subcores=SC_SUBCORES)
    pl.kernel(
        _scatter_body,
        out_type=(),
        mesh=mesh,
        scratch_types=[
            pltpu.VMEM((PER_W // ROWCH, ROWCH), jnp.int32),
            pltpu.VMEM((ROWCH, D), jnp.float32),
            pltpu.SemaphoreType.DMA,
        ],
    )(weighted, idx_flat, out_ref)


JT = 512
NJ = DFF // JT
MT = 1024
NM = NTOK // MT


def _ffn_body(sel_ref, w1_ref, w2_ref, rw_ref, out_ref):
    j = pl.program_id(1)
    p = jnp.dot(sel_ref[...].astype(jnp.bfloat16),
                w1_ref[...].astype(jnp.bfloat16),
                preferred_element_type=jnp.float32)
    pb = jnp.maximum(p, 0.0).astype(jnp.bfloat16)
    d = jnp.dot(pb, w2_ref[...].astype(jnp.bfloat16),
                preferred_element_type=jnp.float32)

    @pl.when(j == 0)
    def _():
        out_ref[...] = d

    @pl.when(jnp.logical_and(j > 0, j < NJ - 1))
    def _():
        out_ref[...] = out_ref[...] + d

    @pl.when(j == NJ - 1)
    def _():
        out_ref[...] = (out_ref[...] + d) * rw_ref[...]


def _ffn(sel, w1, w2, rw_col):
    return pl.pallas_call(
        _ffn_body,
        grid=(NM, NJ),
        in_specs=[
            pl.BlockSpec((MT, D), lambda m, j: (m, 0)),
            pl.BlockSpec((D, JT), lambda m, j: (0, j)),
            pl.BlockSpec((JT, D), lambda m, j: (j, 0)),
            pl.BlockSpec((MT, 1), lambda m, j: (m, 0)),
        ],
        out_specs=pl.BlockSpec((MT, D), lambda m, j: (m, 0)),
        out_shape=jax.ShapeDtypeStruct((NTOK, D), jnp.float32),
        compiler_params=pltpu.CompilerParams(
            dimension_semantics=("arbitrary", "arbitrary")),
    )(sel, w1, w2, rw_col)


def kernel(x, W_r, W1, W2):
    x_flat = x.reshape(B * T, D)
    idx, rw, aux = _router(x_flat, W_r.reshape(1, D))
    idx_flat = idx.reshape(NTOK)
    sel = _gather(x_flat, idx_flat)
    base = _sc_copy(x_flat, sel)
    weighted = _ffn(sel, W1, W2, rw.reshape(NTOK, 1))
    out_ref = jax.new_ref(base)
    _scatter(weighted, idx_flat, out_ref)
    out = jax.freeze(out_ref).reshape(B, T, D)
    return (out, aux.reshape(()))


# speedup vs baseline: 12.0231x; 12.0231x over previous
import functools

import jax
import jax.numpy as jnp
from jax import lax
from jax.experimental import pallas as pl
from jax.experimental.pallas import tpu as pltpu
from jax.experimental.pallas import tpu_sc as plsc

B, T, D, DFF = 4, 4096, 2048, 8192
K = 512
CHUNK = 512
NCHUNK = (B * T) // CHUNK
RC = T // CHUNK
NTOK = B * K
AUX_COEF = 0.01
CAPACITY = 0.125

SC_CORES = 2
SC_SUBCORES = 16
NW = SC_CORES * SC_SUBCORES
PER_W = NTOK // NW
ROWCH = 32

_HI = 0x7FFFFFFF
_SIGN = -2**31


def _flat_cumsum(m2, u_tri, s_blk):
    incl = lax.dot_general(m2, u_tri, (((1,), (0,)), ((), ())),
                           precision=lax.Precision.HIGHEST,
                           preferred_element_type=jnp.float32)
    rowtot = jnp.sum(m2, axis=1, keepdims=True)
    off = lax.dot_general(s_blk, rowtot, (((1,), (0,)), ((), ())),
                          precision=lax.Precision.HIGHEST,
                          preferred_element_type=jnp.float32)
    return incl + off


def _router_body(x_ref, wr_ref, idx_ref, rw_ref, aux_ref, copy_ref,
                 scores_ref):
    i = pl.program_id(0)
    copy_ref[...] = x_ref[...]
    s = lax.dot_general(wr_ref[...].astype(jnp.bfloat16),
                        x_ref[...].astype(jnp.bfloat16),
                        (((1,), (1,)), ((), ())),
                        preferred_element_type=jnp.float32)
    scores_ref[pl.ds(i, 1), :] = s

    @pl.when(i == NCHUNK - 1)
    def _finish():
        s2 = scores_ref[...]
        s3 = s2.reshape(B, RC, CHUNK)
        bits = lax.bitcast_convert_type(s3, jnp.int32)
        key = jnp.where(bits < 0, bits ^ jnp.int32(_HI), bits)

        def _count_ge(thr):
            m = (key >= thr).astype(jnp.int32)
            return jnp.sum(jnp.sum(m, axis=2, keepdims=True), axis=1,
                           keepdims=True)

        def _bit_step(it, prefix_bits):
            cand = prefix_bits | lax.shift_left(jnp.int32(1), 31 - it)
            cnt = _count_ge(cand ^ jnp.int32(_SIGN))
            return jnp.where(cnt >= K, cand, prefix_bits)

        prefix = lax.fori_loop(0, 32, _bit_step,
                               jnp.zeros((B, 1, 1), jnp.int32))
        thr = prefix ^ jnp.int32(_SIGN)
        gt = key > thr
        eq = key == thr
        cnt_gt = jnp.sum(jnp.sum(gt.astype(jnp.int32), axis=2, keepdims=True),
                         axis=1, keepdims=True)
        needed = (K - cnt_gt).astype(jnp.float32)

        lane_i = lax.broadcasted_iota(jnp.int32, (CHUNK, CHUNK), 0)
        lane_ti = lax.broadcasted_iota(jnp.int32, (CHUNK, CHUNK), 1)
        u_tri = (lane_i <= lane_ti).astype(jnp.float32)
        ra = lax.broadcasted_iota(jnp.int32, (NCHUNK, NCHUNK), 0)
        rb = lax.broadcasted_iota(jnp.int32, (NCHUNK, NCHUNK), 1)
        s_blk = ((rb < ra) & ((rb >> 3) == (ra >> 3))).astype(jnp.float32)

        eq2 = eq.reshape(NCHUNK, CHUNK).astype(jnp.float32)
        eq_rank = _flat_cumsum(eq2, u_tri, s_blk).reshape(B, RC, CHUNK)
        selected = gt | (eq & (eq_rank <= needed))
        sel2 = selected.reshape(NCHUNK, CHUNK).astype(jnp.float32)
        sel_cum = _flat_cumsum(sel2, u_tri, s_blk)
        sel_rank = jnp.where(sel2 > 0.0, sel_cum, jnp.float32(0.0))

        jj = (lane_i + 1).astype(jnp.float32)
        lane_row = lax.broadcasted_iota(jnp.int32,
                                        (1, CHUNK), 1).astype(jnp.float32)
        for b in range(B):
            acc = jnp.zeros((2, CHUNK), jnp.float32)
            for r in range(RC):
                rr = b * RC + r
                m_jc = (jj == sel_rank[rr:rr + 1, :]).astype(jnp.float32)
                tid = lane_row + jnp.float32(r * CHUNK + b * T)
                v = jnp.concatenate([tid, s2[rr:rr + 1, :]], axis=0)
                acc = acc + lax.dot_general(
                    v, m_jc, (((1,), (1,)), ((), ())),
                    precision=lax.Precision.HIGHEST,
                    preferred_element_type=jnp.float32)
            idx_ref[b:b + 1, :] = acc[0:1, :].astype(jnp.int32)
            sr = acc[1:2, :]
            e = jnp.exp(sr - jnp.max(sr, axis=1, keepdims=True))
            rw_ref[b:b + 1, :] = e / jnp.sum(e, axis=1, keepdims=True)

        probs = 1.0 / (1.0 + jnp.exp(-s3))
        frac = jnp.sum(jnp.sum(probs, axis=2, keepdims=True), axis=1,
                       keepdims=True) * jnp.float32(1.0 / T)
        dev = (frac - jnp.float32(CAPACITY)) ** 2
        aux_ref[...] = jnp.sum(dev.reshape(B, 1), axis=0,
                               keepdims=True) * jnp.float32(AUX_COEF / B)


def _router(x_flat, wr_row):
    return pl.pallas_call(
        _router_body,
        grid=(NCHUNK,),
        in_specs=[
            pl.BlockSpec((CHUNK, D), lambda i: (i, 0)),
            pl.BlockSpec((1, D), lambda i: (0, 0)),
        ],
        out_specs=[
            pl.BlockSpec((B, K), lambda i: (0, 0)),
            pl.BlockSpec((B, K), lambda i: (0, 0)),
            pl.BlockSpec((1, 1), lambda i: (0, 0)),
            pl.BlockSpec((CHUNK, D), lambda i: (i, 0)),
        ],
        out_shape=[
            jax.ShapeDtypeStruct((B, K), jnp.int32),
            jax.ShapeDtypeStruct((B, K), jnp.float32),
            jax.ShapeDtypeStruct((1, 1), jnp.float32),
            jax.ShapeDtypeStruct((B * T, D), jnp.float32),
        ],
        scratch_shapes=[pltpu.VMEM((NCHUNK, CHUNK), jnp.float32)],
        compiler_params=pltpu.CompilerParams(
            dimension_semantics=("arbitrary",)),
    )(x_flat, wr_row)


def _sc_worker_id():
    return lax.axis_index("s") * SC_CORES + lax.axis_index("c")


def _gather_body(x_hbm, idx_hbm, out_hbm, idxv, rows, sem):
    base = _sc_worker_id() * PER_W
    pltpu.sync_copy(idx_hbm.at[pl.ds(base, PER_W)], idxv)
    for c in range(PER_W // ROWCH):
        pltpu.async_copy(x_hbm.at[idxv.at[pl.ds(c * ROWCH, ROWCH)]], rows,
                         sem).wait()
        pltpu.sync_copy(rows, out_hbm.at[pl.ds(base + c * ROWCH, ROWCH)])


def _gather(x_flat, idx_flat):
    mesh = plsc.VectorSubcoreMesh(core_axis_name="c", subcore_axis_name="s",
                                  num_cores=SC_CORES, num_subcores=SC_SUBCORES)
    return pl.kernel(
        _gather_body,
        out_type=jax.ShapeDtypeStruct((NTOK, D), jnp.float32),
        mesh=mesh,
        scratch_types=[
            pltpu.VMEM((PER_W,), jnp.int32),
            pltpu.VMEM((ROWCH, D), jnp.float32),
            pltpu.SemaphoreType.DMA,
        ],
    )(x_flat, idx_flat)


def _scatter_body(w_hbm, idx_hbm, out_hbm, idxv2, rows, sem):
    base = _sc_worker_id() * PER_W
    for c in range(PER_W // ROWCH):
        pltpu.sync_copy(idx_hbm.at[pl.ds(base + c * ROWCH, ROWCH)],
                        idxv2.at[c])
        pltpu.sync_copy(w_hbm.at[pl.ds(base + c * ROWCH, ROWCH)], rows)
        pltpu.async_copy(rows, out_hbm.at[idxv2.at[c]], sem).wait()


def _scatter(weighted, idx_flat, out_ref):
    mesh = plsc.VectorSubcoreMesh(core_axis_name="c", subcore_axis_name="s",
                                  num_cores=SC_CORES, num_subcores=SC_SUBCORES)
    pl.kernel(
        _scatter_body,
        out_type=(),
        mesh=mesh,
        scratch_types=[
            pltpu.VMEM((PER_W // ROWCH, ROWCH), jnp.int32),
            pltpu.VMEM((ROWCH, D), jnp.float32),
            pltpu.SemaphoreType.DMA,
        ],
    )(weighted, idx_flat, out_ref)


JT = 512
NJ = DFF // JT
MT = 1024
NM = NTOK // MT


def _ffn_body(sel_ref, w1_ref, w2_ref, rw_ref, out_ref):
    j = pl.program_id(1)
    p = jnp.dot(sel_ref[...].astype(jnp.bfloat16),
                w1_ref[...].astype(jnp.bfloat16),
                preferred_element_type=jnp.float32)
    pb = jnp.maximum(p, 0.0).astype(jnp.bfloat16)
    d = jnp.dot(pb, w2_ref[...].astype(jnp.bfloat16),
                preferred_element_type=jnp.float32)

    @pl.when(j == 0)
    def _():
        out_ref[...] = d

    @pl.when(jnp.logical_and(j > 0, j < NJ - 1))
    def _():
        out_ref[...] = out_ref[...] + d

    @pl.when(j == NJ - 1)
    def _():
        out_ref[...] = (out_ref[...] + d) * rw_ref[...]


def _ffn(sel, w1, w2, rw_col):
    return pl.pallas_call(
        _ffn_body,
        grid=(NM, NJ),
        in_specs=[
            pl.BlockSpec((MT, D), lambda m, j: (m, 0)),
            pl.BlockSpec((D, JT), lambda m, j: (0, j)),
            pl.BlockSpec((JT, D), lambda m, j: (j, 0)),
            pl.BlockSpec((MT, 1), lambda m, j: (m, 0)),
        ],
        out_specs=pl.BlockSpec((MT, D), lambda m, j: (m, 0)),
        out_shape=jax.ShapeDtypeStruct((NTOK, D), jnp.float32),
        compiler_params=pltpu.CompilerParams(
            dimension_semantics=("arbitrary", "arbitrary")),
    )(sel, w1, w2, rw_col)


def kernel(x, W_r, W1, W2):
    x_flat = x.reshape(B * T, D)
    idx, rw, aux, base = _router(x_flat, W_r.reshape(1, D))
    idx_flat = idx.reshape(NTOK)
    sel = _gather(x_flat, idx_flat)
    weighted = _ffn(sel, W1, W2, rw.reshape(NTOK, 1))
    out_ref = jax.new_ref(base)
    _scatter(weighted, idx_flat, out_ref)
    out = jax.freeze(out_ref).reshape(B, T, D)
    return (out, aux.reshape(()))
